# baseline (speedup 1.0000x reference)
import functools

import jax
import jax.numpy as jnp
from jax import lax
from jax.experimental import pallas as pl
from jax.experimental.pallas import tpu as pltpu
from jax.experimental.pallas import tpu_sc as plsc

S = 8192
D = 2048
N = S * D
NC = 2
NS = 16
NW = NC * NS
EW = N // NW
CHE = 8192
NCHUNK = EW // CHE
NBUF = 4
NG = NCHUNK // NBUF
L = 16
UNROLL = 8

_mesh = plsc.VectorSubcoreMesh(core_axis_name="c", subcore_axis_name="s")


@functools.partial(
    pl.kernel,
    out_type=jax.ShapeDtypeStruct((N,), jnp.float32),
    mesh=_mesh,
    scratch_types=(
        [pltpu.VMEM((CHE,), jnp.float32) for _ in range(NBUF)]
        + [pltpu.VMEM((CHE,), jnp.float32) for _ in range(NBUF)]
        + [pltpu.SemaphoreType.DMA for _ in range(NBUF)]
        + [pltpu.SemaphoreType.DMA for _ in range(NBUF)]
    ),
)
def _sc_add(x_hbm, t_hbm, out_hbm, *scratch):
    xbufs = scratch[0:NBUF]
    tbufs = scratch[NBUF : 2 * NBUF]
    sins = scratch[2 * NBUF : 3 * NBUF]
    souts = scratch[3 * NBUF : 4 * NBUF]

    wid = lax.axis_index("s") * NC + lax.axis_index("c")
    base = wid * EW

    def issue_in(c, b):
        e0 = base + c * CHE
        pltpu.async_copy(x_hbm.at[pl.ds(e0, CHE)], xbufs[b], sins[b])
        pltpu.async_copy(t_hbm.at[pl.ds(e0, CHE)], tbufs[b], sins[b])

    def wait_in(b):
        pltpu.make_async_copy(x_hbm.at[pl.ds(0, CHE)], xbufs[b], sins[b]).wait()
        pltpu.make_async_copy(t_hbm.at[pl.ds(0, CHE)], tbufs[b], sins[b]).wait()

    def issue_out(c, b):
        e0 = base + c * CHE
        pltpu.async_copy(xbufs[b], out_hbm.at[pl.ds(e0, CHE)], souts[b])

    def wait_out(b):
        pltpu.make_async_copy(xbufs[b], out_hbm.at[pl.ds(0, CHE)], souts[b]).wait()

    def compute(b):
        xb = xbufs[b]
        tb = tbufs[b]

        def body(g2, _):
            off0 = g2 * (L * UNROLL)
            for u in range(UNROLL):
                off = off0 + u * L
                plsc.addupdate(xb.at[pl.ds(off, L)], tb[pl.ds(off, L)])
            return 0

        lax.fori_loop(0, CHE // (L * UNROLL), body, 0)

    issue_in(0, 0)
    issue_in(1, 1)

    def outer(g, _):
        for b in range(NBUF):
            c = g * NBUF + b
            bn = (b + 2) % NBUF
            if b < 2:
                @pl.when(g >= 1)
                def _():
                    wait_out(bn)
                    issue_in(c + 2, bn)

                @pl.when(g == 0)
                def _():
                    issue_in(c + 2, bn)
            else:
                wait_out(bn)

                @pl.when(g < NG - 1)
                def _():
                    issue_in(c + 2, bn)

            wait_in(b)
            compute(b)
            issue_out(c, b)
        return 0

    lax.fori_loop(0, NG, outer, 0)

    wait_out(2)
    wait_out(3)


def kernel(x, pos_table):
    xf = x.reshape(N)
    tf = pos_table[:S].reshape(N)
    return _sc_add(xf, tf).reshape(S, D)
